# baseline (speedup 1.0000x reference)
import functools

import jax
import jax.numpy as jnp
from jax import lax
from jax.experimental import pallas as pl
from jax.experimental.pallas import tpu as pltpu
from jax.experimental.pallas import tpu_sc as plsc

N = 10000
E = 320000
D_IN = 128
D_OUT = 64
EPS = 1e-5

NS = 16
CH = 80
EPT = E // NS
NCH = EPT // CH
NBUF = 5
NPAD = 10240
RPT = NPAD // NS
RB = 40
ROWS_PER_SIGN = E // CH
DST_BASE = 2 * ROWS_PER_SIGN


def _mm_l_body(x_ref, w_ref, h_ref):
    h_ref[...] = jnp.dot(x_ref[...], w_ref[...],
                         preferred_element_type=jnp.float32)


def _mm_l(x, w_l):
    return pl.pallas_call(
        _mm_l_body,
        out_shape=jax.ShapeDtypeStruct((N, D_IN), jnp.float32),
    )(x, w_l)


def _mm_r_body(x_ref, w_ref, r_ref):
    r_ref[...] = jnp.dot(x_ref[...], w_ref[...],
                         preferred_element_type=jnp.float32)


def _mm_r(x, w_r):
    return pl.pallas_call(
        _mm_r_body,
        out_shape=jax.ShapeDtypeStruct((N, D_IN), jnp.float32),
    )(x, w_r)


def _sc_body(h_hbm, idx_hbm, mean_hbm,
             sidx, didx, rows, ones, zb64, zb16, buf64, cbuf,
             accum_sh, cnt_sh,
             *sems):
    c = lax.axis_index("c")
    s = lax.axis_index("s")
    gsems = sems[:NBUF]
    ssems = sems[NBUF:2 * NBUF]
    osem = sems[2 * NBUF]

    zero16 = jnp.zeros((16,), jnp.float32)
    one16 = jnp.ones((16,), jnp.float32)
    for i in range(RB):
        for j in range(4):
            zb64[i, pl.ds(j * 16, 16)] = zero16
        zb16[i, :] = zero16
    for i in range(CH):
        ones[i, :] = one16

    def zloop(i, _):
        r0 = s * RPT + i * RB
        pltpu.sync_copy(zb64, accum_sh.at[pl.ds(r0, RB)])
        pltpu.sync_copy(zb16, cnt_sh.at[pl.ds(r0, RB)])
        return 0

    lax.fori_loop(0, RPT // RB, zloop, 0)
    plsc.subcore_barrier()

    sbase = c * ROWS_PER_SIGN + s * NCH
    dbase = DST_BASE + c * ROWS_PER_SIGN + s * NCH
    pltpu.sync_copy(idx_hbm.at[pl.ds(sbase, NCH)], sidx)
    pltpu.sync_copy(idx_hbm.at[pl.ds(dbase, NCH)], didx)
    for b in range(NBUF):
        pltpu.async_copy(h_hbm.at[sidx.at[b]], rows.at[b], gsems[b])

    def outer(it, _):
        for b in range(NBUF):
            i = it * NBUF + b
            pltpu.make_async_copy(h_hbm.at[sidx.at[b]],
                                  rows.at[b], gsems[b]).wait()
            pltpu.async_copy(rows.at[b], accum_sh.at[didx.at[i]], ssems[b],
                             add=True)
            pltpu.async_copy(ones, cnt_sh.at[didx.at[i]], osem, add=True)
            pltpu.make_async_copy(rows.at[b], accum_sh.at[didx.at[i]],
                                  ssems[b]).wait()
            inext = i + NBUF

            @pl.when(inext < NCH)
            def _():
                pltpu.async_copy(h_hbm.at[sidx.at[inext]], rows.at[b],
                                 gsems[b])
        return 0

    lax.fori_loop(0, NCH // NBUF, outer, 0)

    def drain(i, _):
        pltpu.make_async_copy(ones, cnt_sh.at[didx.at[0]], osem).wait()
        return 0

    lax.fori_loop(0, NCH, drain, 0)
    plsc.subcore_barrier()

    def wloop(i, _):
        r0 = s * RPT + i * RB
        pltpu.sync_copy(accum_sh.at[pl.ds(r0, RB)], buf64)
        pltpu.sync_copy(cnt_sh.at[pl.ds(r0, RB)], cbuf)
        for r in range(RB):
            rec = 1.0 / jnp.maximum(cbuf[r, :], 1.0)
            for g in range(4):
                buf64[r, pl.ds(g * 16, 16)] = (
                    buf64[r, pl.ds(g * 16, 16)] * rec)
        pltpu.sync_copy(buf64, mean_hbm.at[pl.ds(c * NPAD + r0, RB)])
        return 0

    lax.fori_loop(0, RPT // RB, wloop, 0)


_sc_aggregate = functools.partial(
    pl.kernel,
    out_type=jax.ShapeDtypeStruct((2 * NPAD, D_OUT), jnp.float32),
    mesh=plsc.VectorSubcoreMesh(core_axis_name="c", subcore_axis_name="s"),
    compiler_params=pltpu.CompilerParams(use_tc_tiling_on_sc=False),
    scratch_types=(
        pltpu.VMEM((NCH, CH), jnp.int32),
        pltpu.VMEM((NCH, CH), jnp.int32),
        pltpu.VMEM((NBUF, CH, D_OUT), jnp.float32),
        pltpu.VMEM((CH, 16), jnp.float32),
        pltpu.VMEM((RB, D_OUT), jnp.float32),
        pltpu.VMEM((RB, 16), jnp.float32),
        pltpu.VMEM((RB, D_OUT), jnp.float32),
        pltpu.VMEM((RB, 16), jnp.float32),
        pltpu.VMEM_SHARED((NPAD, D_OUT), jnp.float32),
        pltpu.VMEM_SHARED((NPAD, 16), jnp.float32),
    ) + tuple([pltpu.SemaphoreType.DMA] * (2 * NBUF + 1)),
)(_sc_body)


def _finish_body(r_ref, mean_ref, g_ref, be_ref, out_ref):
    m = jnp.concatenate([mean_ref[0:N, :], mean_ref[NPAD:NPAD + N, :]],
                        axis=1)
    y = m + r_ref[...]
    s1 = jnp.sum(y, axis=0, keepdims=True)
    s2 = jnp.sum(y * y, axis=0, keepdims=True)
    mu = s1 * (1.0 / N)
    var = s2 * (1.0 / N) - mu * mu
    out = (y - mu) * jax.lax.rsqrt(var + EPS) * g_ref[...] + be_ref[...]
    out_ref[...] = jnp.maximum(out, 0.0)


def _finish(r, means, g, be):
    return pl.pallas_call(
        _finish_body,
        out_shape=jax.ShapeDtypeStruct((N, 2 * D_OUT), jnp.float32),
    )(r, means, g, be)


def kernel(x, pos_edge_index, neg_edge_index, W_pos_l, W_pos_r, b_pos,
           W_neg_l, W_neg_r, b_neg, gamma, beta):
    w_l = jnp.concatenate([W_pos_l, W_neg_l], axis=1)
    w_r = jnp.concatenate([W_pos_r, W_neg_r], axis=1)
    idx_all = jnp.concatenate([
        pos_edge_index[0].astype(jnp.int32) * 2,
        neg_edge_index[0].astype(jnp.int32) * 2 + 1,
        pos_edge_index[1].astype(jnp.int32),
        neg_edge_index[1].astype(jnp.int32),
    ]).reshape(4 * ROWS_PER_SIGN, CH)
    g = gamma.reshape(1, 2 * D_OUT)
    be = beta.reshape(1, 2 * D_OUT)

    h2 = _mm_l(x, w_l)
    h_flat = h2.reshape(2 * N, D_OUT)
    means = _sc_aggregate(h_flat, idx_all)
    r = _mm_r(x, w_r)
    return _finish(r, means, g, be)


# speedup vs baseline: 1.1605x; 1.1605x over previous
import functools

import jax
import jax.numpy as jnp
from jax import lax
from jax.experimental import pallas as pl
from jax.experimental.pallas import tpu as pltpu
from jax.experimental.pallas import tpu_sc as plsc

N = 10000
E = 320000
D_IN = 128
D_OUT = 64
EPS = 1e-5

NS = 16
CH = 80
EPT = E // NS
NCH = EPT // CH
NBUF = 5
NPAD = 10240
RPT = NPAD // NS
RB = 40
ROWS_PER_SIGN = E // CH
DST_BASE = 2 * ROWS_PER_SIGN


def _mm_l_body(x_ref, w_ref, h_ref):
    h_ref[...] = jnp.dot(x_ref[...], w_ref[...],
                         preferred_element_type=jnp.float32)


def _mm_l(x, w_l):
    return pl.pallas_call(
        _mm_l_body,
        out_shape=jax.ShapeDtypeStruct((N, D_IN), jnp.float32),
    )(x, w_l)


def _mm_r_body(x_ref, w_ref, r_ref):
    r_ref[...] = jnp.dot(x_ref[...], w_ref[...],
                         preferred_element_type=jnp.float32)


def _mm_r(x, w_r):
    return pl.pallas_call(
        _mm_r_body,
        out_shape=jax.ShapeDtypeStruct((N, D_IN), jnp.float32),
    )(x, w_r)


def _sc_body(h_hbm, sp_hbm, sn_hbm, dp_hbm, dn_hbm, mean_hbm,
             sidx, didx, rows, ones, zb64, zb16, buf64, cbuf,
             accum_sh, cnt_sh,
             *sems):
    c = lax.axis_index("c")
    s = lax.axis_index("s")
    gsems = sems[:NBUF]
    ssems = sems[NBUF:2 * NBUF]
    osem = sems[2 * NBUF]

    zero16 = jnp.zeros((16,), jnp.float32)
    one16 = jnp.ones((16,), jnp.float32)
    for i in range(RB):
        for j in range(4):
            zb64[i, pl.ds(j * 16, 16)] = zero16
        zb16[i, :] = zero16
    for i in range(CH):
        ones[i, :] = one16

    def zloop(i, _):
        r0 = s * RPT + i * RB
        pltpu.sync_copy(zb64, accum_sh.at[pl.ds(r0, RB)])
        pltpu.sync_copy(zb16, cnt_sh.at[pl.ds(r0, RB)])
        return 0

    lax.fori_loop(0, RPT // RB, zloop, 0)
    plsc.subcore_barrier()

    tbase = s * NCH

    @pl.when(c == 0)
    def _():
        pltpu.sync_copy(sp_hbm.at[pl.ds(tbase, NCH)], sidx)
        pltpu.sync_copy(dp_hbm.at[pl.ds(tbase, NCH)], didx)

    @pl.when(c == 1)
    def _():
        pltpu.sync_copy(sn_hbm.at[pl.ds(tbase, NCH)], sidx)
        pltpu.sync_copy(dn_hbm.at[pl.ds(tbase, NCH)], didx)
    for b in range(NBUF):
        pltpu.async_copy(h_hbm.at[sidx.at[b]], rows.at[b], gsems[b])

    def outer(it, _):
        for b in range(NBUF):
            i = it * NBUF + b
            pltpu.make_async_copy(h_hbm.at[sidx.at[b]],
                                  rows.at[b], gsems[b]).wait()
            pltpu.async_copy(rows.at[b], accum_sh.at[didx.at[i]], ssems[b],
                             add=True)
            pltpu.async_copy(ones, cnt_sh.at[didx.at[i]], osem, add=True)
            pltpu.make_async_copy(rows.at[b], accum_sh.at[didx.at[i]],
                                  ssems[b]).wait()
            inext = i + NBUF

            @pl.when(inext < NCH)
            def _():
                pltpu.async_copy(h_hbm.at[sidx.at[inext]], rows.at[b],
                                 gsems[b])
        return 0

    lax.fori_loop(0, NCH // NBUF, outer, 0)

    def drain(i, _):
        pltpu.make_async_copy(ones, cnt_sh.at[didx.at[0]], osem).wait()
        return 0

    lax.fori_loop(0, NCH, drain, 0)
    plsc.subcore_barrier()

    def wloop(i, _):
        r0 = s * RPT + i * RB
        pltpu.sync_copy(accum_sh.at[pl.ds(r0, RB)], buf64)
        pltpu.sync_copy(cnt_sh.at[pl.ds(r0, RB)], cbuf)
        for r in range(RB):
            rec = 1.0 / jnp.maximum(cbuf[r, :], 1.0)
            for g in range(4):
                buf64[r, pl.ds(g * 16, 16)] = (
                    buf64[r, pl.ds(g * 16, 16)] * rec)
        pltpu.sync_copy(buf64, mean_hbm.at[pl.ds(c * NPAD + r0, RB)])
        return 0

    lax.fori_loop(0, RPT // RB, wloop, 0)


_sc_aggregate = functools.partial(
    pl.kernel,
    out_type=jax.ShapeDtypeStruct((2 * NPAD, D_OUT), jnp.float32),
    mesh=plsc.VectorSubcoreMesh(core_axis_name="c", subcore_axis_name="s"),
    compiler_params=pltpu.CompilerParams(use_tc_tiling_on_sc=False),
    scratch_types=(
        pltpu.VMEM((NCH, CH), jnp.int32),
        pltpu.VMEM((NCH, CH), jnp.int32),
        pltpu.VMEM((NBUF, CH, D_OUT), jnp.float32),
        pltpu.VMEM((CH, 16), jnp.float32),
        pltpu.VMEM((RB, D_OUT), jnp.float32),
        pltpu.VMEM((RB, 16), jnp.float32),
        pltpu.VMEM((RB, D_OUT), jnp.float32),
        pltpu.VMEM((RB, 16), jnp.float32),
        pltpu.VMEM_SHARED((NPAD, D_OUT), jnp.float32),
        pltpu.VMEM_SHARED((NPAD, 16), jnp.float32),
    ) + tuple([pltpu.SemaphoreType.DMA] * (2 * NBUF + 1)),
)(_sc_body)


def _finish_body(r_ref, mean_ref, g_ref, be_ref, out_ref):
    m = jnp.concatenate([mean_ref[0:N, :], mean_ref[NPAD:NPAD + N, :]],
                        axis=1)
    y = m + r_ref[...]
    s1 = jnp.sum(y, axis=0, keepdims=True)
    s2 = jnp.sum(y * y, axis=0, keepdims=True)
    mu = s1 * (1.0 / N)
    var = s2 * (1.0 / N) - mu * mu
    out = (y - mu) * jax.lax.rsqrt(var + EPS) * g_ref[...] + be_ref[...]
    out_ref[...] = jnp.maximum(out, 0.0)


def _finish(r, means, g, be):
    return pl.pallas_call(
        _finish_body,
        out_shape=jax.ShapeDtypeStruct((N, 2 * D_OUT), jnp.float32),
    )(r, means, g, be)


def kernel(x, pos_edge_index, neg_edge_index, W_pos_l, W_pos_r, b_pos,
           W_neg_l, W_neg_r, b_neg, gamma, beta):
    w_l = jnp.concatenate([W_pos_l, W_neg_l], axis=1)
    w_r = jnp.concatenate([W_pos_r, W_neg_r], axis=1)
    sp = (pos_edge_index[0].astype(jnp.int32) * 2
          ).reshape(ROWS_PER_SIGN, CH)
    sn = (neg_edge_index[0].astype(jnp.int32) * 2 + 1
          ).reshape(ROWS_PER_SIGN, CH)
    dp = pos_edge_index[1].astype(jnp.int32).reshape(ROWS_PER_SIGN, CH)
    dn = neg_edge_index[1].astype(jnp.int32).reshape(ROWS_PER_SIGN, CH)
    g = gamma.reshape(1, 2 * D_OUT)
    be = beta.reshape(1, 2 * D_OUT)

    h2 = _mm_l(x, w_l)
    h_flat = h2.reshape(2 * N, D_OUT)
    means = _sc_aggregate(h_flat, sp, sn, dp, dn)
    r = _mm_r(x, w_r)
    return _finish(r, means, g, be)
